# baseline (speedup 1.0000x reference)
import functools

import jax
import jax.numpy as jnp
import numpy as np
from jax import lax
from jax.experimental import pallas as pl
from jax.experimental.pallas import tpu as pltpu
from jax.experimental.pallas import tpu_sc as plsc

_IN = 128
_T = 32
_NPS = 640
_NPAD = 16 * _NPS
_N0 = 240
_N1 = _NPS - _N0
_CE = 64
_NPC = _CE // _T
_NBUF = 2
_CN = 80
_NMAX = max(_N0, _N1)


def _hq_body(h_ref, qw_ref, qb_ref, o_ref):
    acc = lax.dot_general(h_ref[...], qw_ref[...], (((1,), (1,)), ((), ())),
                          preferred_element_type=jnp.float32)
    o_ref[...] = jnp.maximum(acc + qb_ref[...], 0.0)


def _compute_hq(h, Q_w, Q_b):
    n = h.shape[0]
    blk = 2000
    return pl.pallas_call(
        _hq_body,
        grid=(n // blk,),
        in_specs=[pl.BlockSpec((blk, _IN), lambda i: (i, 0)),
                  pl.BlockSpec((_IN, _IN), lambda i: (0, 0)),
                  pl.BlockSpec((1, _IN), lambda i: (0, 0))],
        out_specs=pl.BlockSpec((blk, _IN), lambda i: (i, 0)),
        out_shape=jax.ShapeDtypeStruct((n, _IN), jnp.float32),
    )(h, Q_w, Q_b.reshape(1, _IN))


def _sc_body(hq_hbm, h_hbm, ns_hbm, nb_hbm, w_hbm,
             agg_hbm, nsh_hbm,
             nb_v, ew_v, rows_v, agg_v, nidx_v, nrows_v,
             sem0, sem1, semn):
    cid = lax.axis_index("c")
    sid = lax.axis_index("s")
    npw = jnp.where(cid == 0, _N0, _N1)
    nbase = sid * _NPS + cid * _N0
    ebase = nbase * _T
    nchunk = npw * _T // _CE
    sems = (sem0, sem1)

    pltpu.sync_copy(nb_hbm.at[pl.ds(ebase, _NMAX * _T)], nb_v)
    pltpu.sync_copy(w_hbm.at[pl.ds(ebase, _NMAX * _T)], ew_v)

    def start(c, b):
        pltpu.async_copy(hq_hbm.at[nb_v.at[pl.ds(c * _CE, _CE)]],
                         rows_v.at[b], sems[b])

    for b0 in range(_NBUF):
        start(b0, b0)

    def process(c, b):
        pltpu.make_async_copy(hq_hbm.at[pl.ds(0, _CE)],
                              rows_v.at[b], sems[b]).wait()
        for j in range(_NPC):
            acc = [jnp.zeros((16,), jnp.float32) for _ in range(8)]
            for t in range(_T):
                e = j * _T + t
                bw = plsc.load_gather(
                    ew_v, [jnp.full((16,), c * _CE + e, jnp.int32)])
                for f in range(8):
                    acc[f] = acc[f] + bw * rows_v[b, e, pl.ds(f * 16, 16)]
            w0 = ew_v[pl.ds(c * _CE + j * _T, 16)]
            w1 = ew_v[pl.ds(c * _CE + j * _T + 16, 16)]
            winv = 1.0 / jnp.broadcast_to(jnp.sum(w0 + w1), (16,))
            for f in range(8):
                agg_v[c * _NPC + j, pl.ds(f * 16, 16)] = acc[f] * winv

    def body(cb, carry):
        c0 = cb * _NBUF
        for b in range(_NBUF):
            c = c0 + b
            process(c, b)

            @pl.when(c + _NBUF < nchunk)
            def _():
                start(c + _NBUF, b)
        return carry

    lax.fori_loop(0, nchunk // _NBUF, body, 0)

    @pl.when(cid == 0)
    def _():
        pltpu.sync_copy(agg_v.at[pl.ds(0, _N0)], agg_hbm.at[pl.ds(nbase, _N0)])

    @pl.when(cid == 1)
    def _():
        pltpu.sync_copy(agg_v.at[pl.ds(0, _N1)], agg_hbm.at[pl.ds(nbase, _N1)])

    def ns_chunk(k, carry):
        noff = nbase + k * _CN
        pltpu.sync_copy(ns_hbm.at[pl.ds(noff, _CN)], nidx_v)
        pltpu.async_copy(h_hbm.at[nidx_v], nrows_v, semn).wait()
        pltpu.sync_copy(nrows_v, nsh_hbm.at[pl.ds(noff, _CN)])
        return carry

    lax.fori_loop(0, npw // _CN, ns_chunk, 0)


def _sc_aggregate(hq, h, ns_p, nb_flat, w_flat):
    mesh = plsc.VectorSubcoreMesh(core_axis_name="c", subcore_axis_name="s")
    f = pl.kernel(
        _sc_body,
        out_type=[jax.ShapeDtypeStruct((_NPAD, _IN), jnp.float32),
                  jax.ShapeDtypeStruct((_NPAD, _IN), jnp.float32)],
        mesh=mesh,
        scratch_types=[
            pltpu.VMEM((_NMAX * _T,), jnp.int32),
            pltpu.VMEM((_NMAX * _T,), jnp.float32),
            pltpu.VMEM((_NBUF, _CE, _IN), jnp.float32),
            pltpu.VMEM((_NMAX, _IN), jnp.float32),
            pltpu.VMEM((_CN,), jnp.int32),
            pltpu.VMEM((_CN, _IN), jnp.float32),
            pltpu.SemaphoreType.DMA,
            pltpu.SemaphoreType.DMA,
            pltpu.SemaphoreType.DMA,
        ],
        compiler_params=pltpu.CompilerParams(needs_layout_passes=False),
    )
    return f(hq, h, ns_p, nb_flat, w_flat)


def _out_body(nsh_ref, agg_ref, w_ref, wb_ref, o_ref):
    w = w_ref[...]
    x = lax.dot_general(nsh_ref[...], w[:, :_IN], (((1,), (1,)), ((), ())),
                        preferred_element_type=jnp.float32)
    x = x + lax.dot_general(agg_ref[...], w[:, _IN:], (((1,), (1,)), ((), ())),
                            preferred_element_type=jnp.float32)
    x = jnp.maximum(x + wb_ref[...], 0.0)
    nrm = jnp.sqrt(jnp.sum(x * x, axis=1, keepdims=True))
    o_ref[...] = x / nrm


def _compute_out(nsh, agg, W_w, W_b):
    n = nsh.shape[0]
    blk = 1000
    return pl.pallas_call(
        _out_body,
        grid=(n // blk,),
        in_specs=[pl.BlockSpec((blk, _IN), lambda i: (i, 0)),
                  pl.BlockSpec((blk, _IN), lambda i: (i, 0)),
                  pl.BlockSpec((_IN, 2 * _IN), lambda i: (0, 0)),
                  pl.BlockSpec((1, _IN), lambda i: (0, 0))],
        out_specs=pl.BlockSpec((blk, _IN), lambda i: (i, 0)),
        out_shape=jax.ShapeDtypeStruct((n, _IN), jnp.float32),
    )(nsh, agg, W_w, W_b.reshape(1, _IN))


def kernel(h, nodeset, nb_nodes, nb_weights, Q_w, Q_b, W_w, W_b):
    n_set = nodeset.shape[0]
    hq = _compute_hq(h, Q_w, Q_b)
    pad = _NPAD - n_set
    epad = pad + _NMAX - min(_N0, _N1)
    ns_p = jnp.concatenate(
        [nodeset.astype(jnp.int32), jnp.zeros((pad,), jnp.int32)])
    nb_flat = jnp.concatenate(
        [nb_nodes.astype(jnp.int32),
         jnp.zeros((epad, _T), jnp.int32)]).reshape(-1)
    w_flat = jnp.concatenate(
        [nb_weights, jnp.ones((epad, _T), jnp.float32)]).reshape(-1)
    agg, nsh = _sc_aggregate(hq, h, ns_p, nb_flat, w_flat)
    return _compute_out(nsh[:n_set], agg[:n_set], W_w, W_b)


# speedup vs baseline: 1.1706x; 1.1706x over previous
import functools

import jax
import jax.numpy as jnp
import numpy as np
from jax import lax
from jax.experimental import pallas as pl
from jax.experimental.pallas import tpu as pltpu
from jax.experimental.pallas import tpu_sc as plsc

_IN = 128
_T = 32
_NPS = 640
_NPAD = 16 * _NPS
_N0 = 432
_N1 = _NPS - _N0
_CE = 64
_NPC = _CE // _T
_NBUF = 2
_CN = 80
_NMAX = max(_N0, _N1)


def _hq_body(h_ref, qw_ref, qb_ref, o_ref):
    acc = lax.dot_general(h_ref[...], qw_ref[...], (((1,), (1,)), ((), ())),
                          preferred_element_type=jnp.float32)
    o_ref[...] = jnp.maximum(acc + qb_ref[...], 0.0)


def _compute_hq(h, Q_w, Q_b):
    n = h.shape[0]
    blk = 2000
    return pl.pallas_call(
        _hq_body,
        grid=(n // blk,),
        in_specs=[pl.BlockSpec((blk, _IN), lambda i: (i, 0)),
                  pl.BlockSpec((_IN, _IN), lambda i: (0, 0)),
                  pl.BlockSpec((1, _IN), lambda i: (0, 0))],
        out_specs=pl.BlockSpec((blk, _IN), lambda i: (i, 0)),
        out_shape=jax.ShapeDtypeStruct((n, _IN), jnp.float32),
    )(h, Q_w, Q_b.reshape(1, _IN))


def _sc_body(hq_hbm, h_hbm, ns_hbm, nb_hbm, w_hbm,
             agg_hbm, nsh_hbm,
             nb_v, ew_v, rows_v, agg_v, nidx_v, nrows_v,
             sem0, sem1, semn):
    cid = lax.axis_index("c")
    sid = lax.axis_index("s")
    npw = jnp.where(cid == 0, _N0, _N1)
    nbase = sid * _NPS + cid * _N0
    ebase = nbase * _T
    nchunk = npw * _T // _CE
    sems = (sem0, sem1)

    pltpu.sync_copy(nb_hbm.at[pl.ds(ebase, _NMAX * _T)], nb_v)
    pltpu.sync_copy(w_hbm.at[pl.ds(ebase, _NMAX * _T)], ew_v)

    def start(c, b):
        pltpu.async_copy(hq_hbm.at[nb_v.at[pl.ds(c * _CE, _CE)]],
                         rows_v.at[b], sems[b])

    for b0 in range(_NBUF):
        start(b0, b0)

    def process(c, b):
        pltpu.make_async_copy(hq_hbm.at[pl.ds(0, _CE)],
                              rows_v.at[b], sems[b]).wait()
        for j in range(_NPC):
            acc = [jnp.zeros((16,), jnp.float32) for _ in range(8)]
            for t in range(_T):
                e = j * _T + t
                bw = plsc.load_gather(
                    ew_v, [jnp.full((16,), c * _CE + e, jnp.int32)])
                for f in range(8):
                    acc[f] = acc[f] + bw * rows_v[b, e, pl.ds(f * 16, 16)]
            w0 = ew_v[pl.ds(c * _CE + j * _T, 16)]
            w1 = ew_v[pl.ds(c * _CE + j * _T + 16, 16)]
            winv = 1.0 / jnp.broadcast_to(jnp.sum(w0 + w1), (16,))
            for f in range(8):
                agg_v[c * _NPC + j, pl.ds(f * 16, 16)] = acc[f] * winv

    def body(cb, carry):
        c0 = cb * _NBUF
        for b in range(_NBUF):
            c = c0 + b
            process(c, b)

            @pl.when(c + _NBUF < nchunk)
            def _():
                start(c + _NBUF, b)
        return carry

    lax.fori_loop(0, nchunk // _NBUF, body, 0)

    @pl.when(cid == 0)
    def _():
        pltpu.sync_copy(agg_v.at[pl.ds(0, _N0)], agg_hbm.at[pl.ds(nbase, _N0)])

    @pl.when(cid == 1)
    def _():
        pltpu.sync_copy(agg_v.at[pl.ds(0, _N1)], agg_hbm.at[pl.ds(nbase, _N1)])

    wid = sid * 2 + cid
    nsym = _NPAD // 32

    def ns_chunk(k, carry):
        noff = wid * nsym + k * _CN
        pltpu.sync_copy(ns_hbm.at[pl.ds(noff, _CN)], nidx_v)
        pltpu.async_copy(h_hbm.at[nidx_v], nrows_v, semn).wait()
        pltpu.sync_copy(nrows_v, nsh_hbm.at[pl.ds(noff, _CN)])
        return carry

    lax.fori_loop(0, nsym // _CN, ns_chunk, 0)


def _sc_aggregate(hq, h, ns_p, nb_flat, w_flat):
    mesh = plsc.VectorSubcoreMesh(core_axis_name="c", subcore_axis_name="s")
    f = pl.kernel(
        _sc_body,
        out_type=[jax.ShapeDtypeStruct((_NPAD, _IN), jnp.float32),
                  jax.ShapeDtypeStruct((_NPAD, _IN), jnp.float32)],
        mesh=mesh,
        scratch_types=[
            pltpu.VMEM((_NMAX * _T,), jnp.int32),
            pltpu.VMEM((_NMAX * _T,), jnp.float32),
            pltpu.VMEM((_NBUF, _CE, _IN), jnp.float32),
            pltpu.VMEM((_NMAX, _IN), jnp.float32),
            pltpu.VMEM((_CN,), jnp.int32),
            pltpu.VMEM((_CN, _IN), jnp.float32),
            pltpu.SemaphoreType.DMA,
            pltpu.SemaphoreType.DMA,
            pltpu.SemaphoreType.DMA,
        ],
        compiler_params=pltpu.CompilerParams(needs_layout_passes=False),
    )
    return f(hq, h, ns_p, nb_flat, w_flat)


def _out_body(nsh_ref, agg_ref, w_ref, wb_ref, o_ref):
    w = w_ref[...]
    x = lax.dot_general(nsh_ref[...], w[:, :_IN], (((1,), (1,)), ((), ())),
                        preferred_element_type=jnp.float32)
    x = x + lax.dot_general(agg_ref[...], w[:, _IN:], (((1,), (1,)), ((), ())),
                            preferred_element_type=jnp.float32)
    x = jnp.maximum(x + wb_ref[...], 0.0)
    nrm = jnp.sqrt(jnp.sum(x * x, axis=1, keepdims=True))
    o_ref[...] = x / nrm


def _compute_out(nsh, agg, W_w, W_b):
    n = nsh.shape[0]
    blk = 1000
    return pl.pallas_call(
        _out_body,
        grid=(n // blk,),
        in_specs=[pl.BlockSpec((blk, _IN), lambda i: (i, 0)),
                  pl.BlockSpec((blk, _IN), lambda i: (i, 0)),
                  pl.BlockSpec((_IN, 2 * _IN), lambda i: (0, 0)),
                  pl.BlockSpec((1, _IN), lambda i: (0, 0))],
        out_specs=pl.BlockSpec((blk, _IN), lambda i: (i, 0)),
        out_shape=jax.ShapeDtypeStruct((n, _IN), jnp.float32),
    )(nsh, agg, W_w, W_b.reshape(1, _IN))


def kernel(h, nodeset, nb_nodes, nb_weights, Q_w, Q_b, W_w, W_b):
    n_set = nodeset.shape[0]
    hq = _compute_hq(h, Q_w, Q_b)
    pad = _NPAD - n_set
    epad = pad + _NMAX - min(_N0, _N1)
    ns_p = jnp.concatenate(
        [nodeset.astype(jnp.int32), jnp.zeros((pad,), jnp.int32)])
    nb_flat = jnp.concatenate(
        [nb_nodes.astype(jnp.int32),
         jnp.zeros((epad, _T), jnp.int32)]).reshape(-1)
    w_flat = jnp.concatenate(
        [nb_weights, jnp.ones((epad, _T), jnp.float32)]).reshape(-1)
    agg, nsh = _sc_aggregate(hq, h, ns_p, nb_flat, w_flat)
    return _compute_out(nsh[:n_set], agg[:n_set], W_w, W_b)


# speedup vs baseline: 1.1749x; 1.0036x over previous
import functools

import jax
import jax.numpy as jnp
import numpy as np
from jax import lax
from jax.experimental import pallas as pl
from jax.experimental.pallas import tpu as pltpu
from jax.experimental.pallas import tpu_sc as plsc

_IN = 128
_T = 32
_NPS = 640
_NPAD = 16 * _NPS
_N0 = 432
_N1 = _NPS - _N0
_CE = 64
_NPC = _CE // _T
_NBUF = 2
_CN = 80
_NMAX = max(_N0, _N1)


def _hq_body(h_ref, qw_ref, qb_ref, o_ref):
    acc = lax.dot_general(h_ref[...], qw_ref[...], (((1,), (1,)), ((), ())),
                          preferred_element_type=jnp.float32)
    o_ref[...] = jnp.maximum(acc + qb_ref[...], 0.0)


def _compute_hq(h, Q_w, Q_b):
    n = h.shape[0]
    blk = 2000
    return pl.pallas_call(
        _hq_body,
        grid=(n // blk,),
        in_specs=[pl.BlockSpec((blk, _IN), lambda i: (i, 0)),
                  pl.BlockSpec((_IN, _IN), lambda i: (0, 0)),
                  pl.BlockSpec((1, _IN), lambda i: (0, 0))],
        out_specs=pl.BlockSpec((blk, _IN), lambda i: (i, 0)),
        out_shape=jax.ShapeDtypeStruct((n, _IN), jnp.float32),
    )(h, Q_w, Q_b.reshape(1, _IN))


def _sc_body(hq_hbm, h_hbm, ns_hbm, nb_hbm, w_hbm,
             agg_hbm, nsh_hbm,
             nb_v, ew_v, rows_v, agg_v, nidx_v, nrows_v,
             sem0, sem1, semn):
    cid = lax.axis_index("c")
    sid = lax.axis_index("s")
    npw = jnp.where(cid == 0, _N0, _N1)
    nbase = sid * _NPS + cid * _N0
    ebase = nbase * _T
    nchunk = npw * _T // _CE
    sems = (sem0, sem1)

    pltpu.sync_copy(nb_hbm.at[pl.ds(ebase, _NMAX * _T)], nb_v)
    pltpu.sync_copy(w_hbm.at[pl.ds(ebase, _NMAX * _T)], ew_v)

    def start(c, b):
        pltpu.async_copy(hq_hbm.at[nb_v.at[pl.ds(c * _CE, _CE)]],
                         rows_v.at[b], sems[b], priority=b % 2)

    for b0 in range(_NBUF):
        start(b0, b0)

    def process(c, b):
        pltpu.make_async_copy(hq_hbm.at[pl.ds(0, _CE)],
                              rows_v.at[b], sems[b]).wait()
        for j in range(_NPC):
            acc = [jnp.zeros((16,), jnp.float32) for _ in range(8)]
            for t in range(_T):
                e = j * _T + t
                bw = plsc.load_gather(
                    ew_v, [jnp.full((16,), c * _CE + e, jnp.int32)])
                for f in range(8):
                    acc[f] = acc[f] + bw * rows_v[b, e, pl.ds(f * 16, 16)]
            w0 = ew_v[pl.ds(c * _CE + j * _T, 16)]
            w1 = ew_v[pl.ds(c * _CE + j * _T + 16, 16)]
            winv = 1.0 / jnp.broadcast_to(jnp.sum(w0 + w1), (16,))
            for f in range(8):
                agg_v[c * _NPC + j, pl.ds(f * 16, 16)] = acc[f] * winv

    def body(cb, carry):
        c0 = cb * _NBUF
        for b in range(_NBUF):
            c = c0 + b
            process(c, b)

            @pl.when(c + _NBUF < nchunk)
            def _():
                start(c + _NBUF, b)
        return carry

    lax.fori_loop(0, nchunk // _NBUF, body, 0)

    @pl.when(cid == 0)
    def _():
        pltpu.sync_copy(agg_v.at[pl.ds(0, _N0)], agg_hbm.at[pl.ds(nbase, _N0)])

    @pl.when(cid == 1)
    def _():
        pltpu.sync_copy(agg_v.at[pl.ds(0, _N1)], agg_hbm.at[pl.ds(nbase, _N1)])

    wid = sid * 2 + cid
    nsym = _NPAD // 32

    def ns_chunk(k, carry):
        noff = wid * nsym + k * _CN
        pltpu.sync_copy(ns_hbm.at[pl.ds(noff, _CN)], nidx_v)
        pltpu.async_copy(h_hbm.at[nidx_v], nrows_v, semn).wait()
        pltpu.sync_copy(nrows_v, nsh_hbm.at[pl.ds(noff, _CN)])
        return carry

    lax.fori_loop(0, nsym // _CN, ns_chunk, 0)


def _sc_aggregate(hq, h, ns_p, nb_flat, w_flat):
    mesh = plsc.VectorSubcoreMesh(core_axis_name="c", subcore_axis_name="s")
    f = pl.kernel(
        _sc_body,
        out_type=[jax.ShapeDtypeStruct((_NPAD, _IN), jnp.float32),
                  jax.ShapeDtypeStruct((_NPAD, _IN), jnp.float32)],
        mesh=mesh,
        scratch_types=[
            pltpu.VMEM((_NMAX * _T,), jnp.int32),
            pltpu.VMEM((_NMAX * _T,), jnp.float32),
            pltpu.VMEM((_NBUF, _CE, _IN), jnp.float32),
            pltpu.VMEM((_NMAX, _IN), jnp.float32),
            pltpu.VMEM((_CN,), jnp.int32),
            pltpu.VMEM((_CN, _IN), jnp.float32),
            pltpu.SemaphoreType.DMA,
            pltpu.SemaphoreType.DMA,
            pltpu.SemaphoreType.DMA,
        ],
        compiler_params=pltpu.CompilerParams(needs_layout_passes=False),
    )
    return f(hq, h, ns_p, nb_flat, w_flat)


def _out_body(nsh_ref, agg_ref, w_ref, wb_ref, o_ref):
    w = w_ref[...]
    x = lax.dot_general(nsh_ref[...], w[:, :_IN], (((1,), (1,)), ((), ())),
                        preferred_element_type=jnp.float32)
    x = x + lax.dot_general(agg_ref[...], w[:, _IN:], (((1,), (1,)), ((), ())),
                            preferred_element_type=jnp.float32)
    x = jnp.maximum(x + wb_ref[...], 0.0)
    nrm = jnp.sqrt(jnp.sum(x * x, axis=1, keepdims=True))
    o_ref[...] = x / nrm


def _compute_out(nsh, agg, W_w, W_b):
    n = nsh.shape[0]
    blk = 1000
    return pl.pallas_call(
        _out_body,
        grid=(n // blk,),
        in_specs=[pl.BlockSpec((blk, _IN), lambda i: (i, 0)),
                  pl.BlockSpec((blk, _IN), lambda i: (i, 0)),
                  pl.BlockSpec((_IN, 2 * _IN), lambda i: (0, 0)),
                  pl.BlockSpec((1, _IN), lambda i: (0, 0))],
        out_specs=pl.BlockSpec((blk, _IN), lambda i: (i, 0)),
        out_shape=jax.ShapeDtypeStruct((n, _IN), jnp.float32),
    )(nsh, agg, W_w, W_b.reshape(1, _IN))


def kernel(h, nodeset, nb_nodes, nb_weights, Q_w, Q_b, W_w, W_b):
    n_set = nodeset.shape[0]
    hq = _compute_hq(h, Q_w, Q_b)
    pad = _NPAD - n_set
    epad = pad + _NMAX - min(_N0, _N1)
    ns_p = jnp.concatenate(
        [nodeset.astype(jnp.int32), jnp.zeros((pad,), jnp.int32)])
    nb_flat = jnp.concatenate(
        [nb_nodes.astype(jnp.int32),
         jnp.zeros((epad, _T), jnp.int32)]).reshape(-1)
    w_flat = jnp.concatenate(
        [nb_weights, jnp.ones((epad, _T), jnp.float32)]).reshape(-1)
    agg, nsh = _sc_aggregate(hq, h, ns_p, nb_flat, w_flat)
    return _compute_out(nsh[:n_set], agg[:n_set], W_w, W_b)


# speedup vs baseline: 1.1791x; 1.0036x over previous
import functools

import jax
import jax.numpy as jnp
import numpy as np
from jax import lax
from jax.experimental import pallas as pl
from jax.experimental.pallas import tpu as pltpu
from jax.experimental.pallas import tpu_sc as plsc

_IN = 128
_T = 32
_NPS = 640
_NPAD = 16 * _NPS
_N0 = 432
_N1 = _NPS - _N0
_CE = 64
_NPC = _CE // _T
_NBUF = 2
_CN = 80
_NMAX = max(_N0, _N1)


def _hq_body(h_ref, qw_ref, qb_ref, o_ref):
    acc = lax.dot_general(h_ref[...], qw_ref[...], (((1,), (1,)), ((), ())),
                          preferred_element_type=jnp.float32)
    o_ref[...] = jnp.maximum(acc + qb_ref[...], 0.0)


def _compute_hq(h, Q_w, Q_b):
    n = h.shape[0]
    blk = 2000
    return pl.pallas_call(
        _hq_body,
        grid=(n // blk,),
        in_specs=[pl.BlockSpec((blk, _IN), lambda i: (i, 0)),
                  pl.BlockSpec((_IN, _IN), lambda i: (0, 0)),
                  pl.BlockSpec((1, _IN), lambda i: (0, 0))],
        out_specs=pl.BlockSpec((blk, _IN), lambda i: (i, 0)),
        out_shape=jax.ShapeDtypeStruct((n, _IN), jnp.float32),
    )(h, Q_w, Q_b.reshape(1, _IN))


def _sc_body(hq_hbm, nb_hbm, w_hbm, agg_hbm,
             nb_v, ew_v, rows_v, agg_v, sem0, sem1):
    cid = lax.axis_index("c")
    sid = lax.axis_index("s")
    npw = jnp.where(cid == 0, _N0, _N1)
    nbase = sid * _NPS + cid * _N0
    ebase = nbase * _T
    nchunk = npw * _T // _CE
    sems = (sem0, sem1)

    pltpu.sync_copy(nb_hbm.at[pl.ds(ebase, _NMAX * _T)], nb_v)
    pltpu.sync_copy(w_hbm.at[pl.ds(ebase, _NMAX * _T)], ew_v)

    def start(c, b):
        pltpu.async_copy(hq_hbm.at[nb_v.at[pl.ds(c * _CE, _CE)]],
                         rows_v.at[b], sems[b], priority=b % 2)

    for b0 in range(_NBUF):
        start(b0, b0)

    def process(c, b):
        pltpu.make_async_copy(hq_hbm.at[pl.ds(0, _CE)],
                              rows_v.at[b], sems[b]).wait()
        for j in range(_NPC):
            acc = [jnp.zeros((16,), jnp.float32) for _ in range(8)]
            for t in range(_T):
                e = j * _T + t
                bw = plsc.load_gather(
                    ew_v, [jnp.full((16,), c * _CE + e, jnp.int32)])
                for f in range(8):
                    acc[f] = acc[f] + bw * rows_v[b, e, pl.ds(f * 16, 16)]
            w0 = ew_v[pl.ds(c * _CE + j * _T, 16)]
            w1 = ew_v[pl.ds(c * _CE + j * _T + 16, 16)]
            winv = 1.0 / jnp.broadcast_to(jnp.sum(w0 + w1), (16,))
            for f in range(8):
                agg_v[c * _NPC + j, pl.ds(f * 16, 16)] = acc[f] * winv

    def body(cb, carry):
        c0 = cb * _NBUF
        for b in range(_NBUF):
            c = c0 + b
            process(c, b)

            @pl.when(c + _NBUF < nchunk)
            def _():
                start(c + _NBUF, b)
        return carry

    lax.fori_loop(0, nchunk // _NBUF, body, 0)

    @pl.when(cid == 0)
    def _():
        pltpu.sync_copy(agg_v.at[pl.ds(0, _N0)], agg_hbm.at[pl.ds(nbase, _N0)])

    @pl.when(cid == 1)
    def _():
        pltpu.sync_copy(agg_v.at[pl.ds(0, _N1)], agg_hbm.at[pl.ds(nbase, _N1)])


def _sc_aggregate(hq, nb_flat, w_flat):
    mesh = plsc.VectorSubcoreMesh(core_axis_name="c", subcore_axis_name="s")
    f = pl.kernel(
        _sc_body,
        out_type=jax.ShapeDtypeStruct((_NPAD, _IN), jnp.float32),
        mesh=mesh,
        scratch_types=[
            pltpu.VMEM((_NMAX * _T,), jnp.int32),
            pltpu.VMEM((_NMAX * _T,), jnp.float32),
            pltpu.VMEM((_NBUF, _CE, _IN), jnp.float32),
            pltpu.VMEM((_NMAX, _IN), jnp.float32),
            pltpu.SemaphoreType.DMA,
            pltpu.SemaphoreType.DMA,
        ],
        compiler_params=pltpu.CompilerParams(needs_layout_passes=False),
    )
    return f(hq, nb_flat, w_flat)


def _ns_body(h_hbm, ns_hbm, nsh_hbm, nidx_v, nrows_v, semn):
    wid = lax.axis_index("s") * 2 + lax.axis_index("c")
    nsym = _NPAD // 32

    def ns_chunk(k, carry):
        noff = wid * nsym + k * _CN
        pltpu.sync_copy(ns_hbm.at[pl.ds(noff, _CN)], nidx_v)
        pltpu.async_copy(h_hbm.at[nidx_v], nrows_v, semn).wait()
        pltpu.sync_copy(nrows_v, nsh_hbm.at[pl.ds(noff, _CN)])
        return carry

    lax.fori_loop(0, nsym // _CN, ns_chunk, 0)


def _ns_gather(h, ns_p):
    mesh = plsc.VectorSubcoreMesh(core_axis_name="c", subcore_axis_name="s")
    f = pl.kernel(
        _ns_body,
        out_type=jax.ShapeDtypeStruct((_NPAD, _IN), jnp.float32),
        mesh=mesh,
        scratch_types=[
            pltpu.VMEM((_CN,), jnp.int32),
            pltpu.VMEM((_CN, _IN), jnp.float32),
            pltpu.SemaphoreType.DMA,
        ],
        compiler_params=pltpu.CompilerParams(needs_layout_passes=False),
    )
    return f(h, ns_p)


def _out_body(nsh_ref, agg_ref, w_ref, wb_ref, o_ref):
    w = w_ref[...]
    x = lax.dot_general(nsh_ref[...], w[:, :_IN], (((1,), (1,)), ((), ())),
                        preferred_element_type=jnp.float32)
    x = x + lax.dot_general(agg_ref[...], w[:, _IN:], (((1,), (1,)), ((), ())),
                            preferred_element_type=jnp.float32)
    x = jnp.maximum(x + wb_ref[...], 0.0)
    nrm = jnp.sqrt(jnp.sum(x * x, axis=1, keepdims=True))
    o_ref[...] = x / nrm


def _compute_out(nsh, agg, W_w, W_b):
    n = nsh.shape[0]
    blk = 1000
    return pl.pallas_call(
        _out_body,
        grid=(n // blk,),
        in_specs=[pl.BlockSpec((blk, _IN), lambda i: (i, 0)),
                  pl.BlockSpec((blk, _IN), lambda i: (i, 0)),
                  pl.BlockSpec((_IN, 2 * _IN), lambda i: (0, 0)),
                  pl.BlockSpec((1, _IN), lambda i: (0, 0))],
        out_specs=pl.BlockSpec((blk, _IN), lambda i: (i, 0)),
        out_shape=jax.ShapeDtypeStruct((n, _IN), jnp.float32),
    )(nsh, agg, W_w, W_b.reshape(1, _IN))


def kernel(h, nodeset, nb_nodes, nb_weights, Q_w, Q_b, W_w, W_b):
    n_set = nodeset.shape[0]
    hq = _compute_hq(h, Q_w, Q_b)
    pad = _NPAD - n_set
    epad = pad + _NMAX - min(_N0, _N1)
    ns_p = jnp.concatenate(
        [nodeset.astype(jnp.int32), jnp.zeros((pad,), jnp.int32)])
    nb_flat = jnp.concatenate(
        [nb_nodes.astype(jnp.int32),
         jnp.zeros((epad, _T), jnp.int32)]).reshape(-1)
    w_flat = jnp.concatenate(
        [nb_weights, jnp.ones((epad, _T), jnp.float32)]).reshape(-1)
    nsh = _ns_gather(h, ns_p)
    agg = _sc_aggregate(hq, nb_flat, w_flat)
    return _compute_out(nsh[:n_set], agg[:n_set], W_w, W_b)


# speedup vs baseline: 1.2333x; 1.0459x over previous
import functools

import jax
import jax.numpy as jnp
import numpy as np
from jax import lax
from jax.experimental import pallas as pl
from jax.experimental.pallas import tpu as pltpu
from jax.experimental.pallas import tpu_sc as plsc

_IN = 128
_T = 32
_NPS = 640
_NPAD = 16 * _NPS
_N0 = 424
_N1 = _NPS - _N0
_CE = 64
_NPC = _CE // _T
_NBUF = 2
_CN = 80
_NMAX = max(_N0, _N1)


def _hq_body(h_ref, qw_ref, qb_ref, o_ref):
    acc = lax.dot_general(h_ref[...], qw_ref[...], (((1,), (1,)), ((), ())),
                          preferred_element_type=jnp.float32)
    o_ref[...] = jnp.maximum(acc + qb_ref[...], 0.0)


def _compute_hq(h, Q_w, Q_b):
    n = h.shape[0]
    blk = 4000
    return pl.pallas_call(
        _hq_body,
        grid=(n // blk,),
        in_specs=[pl.BlockSpec((blk, _IN), lambda i: (i, 0)),
                  pl.BlockSpec((_IN, _IN), lambda i: (0, 0)),
                  pl.BlockSpec((1, _IN), lambda i: (0, 0))],
        out_specs=pl.BlockSpec((blk, _IN), lambda i: (i, 0)),
        out_shape=jax.ShapeDtypeStruct((n, _IN), jnp.float32),
    )(h, Q_w, Q_b.reshape(1, _IN))


def _sc_body(hq_hbm, nb_hbm, w_hbm, agg_hbm,
             nb_v, ew_v, rows_v, agg_v, sem0, sem1):
    cid = lax.axis_index("c")
    sid = lax.axis_index("s")
    npw = jnp.where(cid == 0, _N0, _N1)
    nbase = sid * _NPS + cid * _N0
    ebase = nbase * _T
    nchunk = npw * _T // _CE
    sems = (sem0, sem1)

    pltpu.sync_copy(nb_hbm.at[pl.ds(ebase, _NMAX * _T)], nb_v)
    pltpu.sync_copy(w_hbm.at[pl.ds(ebase, _NMAX * _T)], ew_v)

    def start(c, b):
        pltpu.async_copy(hq_hbm.at[nb_v.at[pl.ds(c * _CE, _CE)]],
                         rows_v.at[b], sems[b], priority=b % 2)

    for b0 in range(_NBUF):
        start(b0, b0)

    def process(c, b):
        pltpu.make_async_copy(hq_hbm.at[pl.ds(0, _CE)],
                              rows_v.at[b], sems[b]).wait()
        for j in range(_NPC):
            acc = [jnp.zeros((16,), jnp.float32) for _ in range(8)]
            for t in range(_T):
                e = j * _T + t
                bw = plsc.load_gather(
                    ew_v, [jnp.full((16,), c * _CE + e, jnp.int32)])
                for f in range(8):
                    acc[f] = acc[f] + bw * rows_v[b, e, pl.ds(f * 16, 16)]
            w0 = ew_v[pl.ds(c * _CE + j * _T, 16)]
            w1 = ew_v[pl.ds(c * _CE + j * _T + 16, 16)]
            winv = 1.0 / jnp.broadcast_to(jnp.sum(w0 + w1), (16,))
            for f in range(8):
                agg_v[c * _NPC + j, pl.ds(f * 16, 16)] = acc[f] * winv

    def body(cb, carry):
        c0 = cb * _NBUF
        for b in range(_NBUF):
            c = c0 + b
            process(c, b)

            @pl.when(c + _NBUF < nchunk)
            def _():
                start(c + _NBUF, b)
        return carry

    lax.fori_loop(0, nchunk // _NBUF, body, 0)

    @pl.when(cid == 0)
    def _():
        pltpu.sync_copy(agg_v.at[pl.ds(0, _N0)], agg_hbm.at[pl.ds(nbase, _N0)])

    @pl.when(cid == 1)
    def _():
        pltpu.sync_copy(agg_v.at[pl.ds(0, _N1)], agg_hbm.at[pl.ds(nbase, _N1)])


def _sc_aggregate(hq, nb_flat, w_flat):
    mesh = plsc.VectorSubcoreMesh(core_axis_name="c", subcore_axis_name="s")
    f = pl.kernel(
        _sc_body,
        out_type=jax.ShapeDtypeStruct((_NPAD, _IN), jnp.float32),
        mesh=mesh,
        scratch_types=[
            pltpu.VMEM((_NMAX * _T,), jnp.int32),
            pltpu.VMEM((_NMAX * _T,), jnp.float32),
            pltpu.VMEM((_NBUF, _CE, _IN), jnp.float32),
            pltpu.VMEM((_NMAX, _IN), jnp.float32),
            pltpu.SemaphoreType.DMA,
            pltpu.SemaphoreType.DMA,
        ],
        compiler_params=pltpu.CompilerParams(needs_layout_passes=False),
    )
    return f(hq, nb_flat, w_flat)


def _ns_body(h_hbm, ns_hbm, nsh_hbm, nidx_v, nrows_v, semn):
    wid = lax.axis_index("s") * 2 + lax.axis_index("c")
    nsym = _NPAD // 32

    def ns_chunk(k, carry):
        noff = wid * nsym + k * _CN
        pltpu.sync_copy(ns_hbm.at[pl.ds(noff, _CN)], nidx_v)
        pltpu.async_copy(h_hbm.at[nidx_v], nrows_v, semn).wait()
        pltpu.sync_copy(nrows_v, nsh_hbm.at[pl.ds(noff, _CN)])
        return carry

    lax.fori_loop(0, nsym // _CN, ns_chunk, 0)


def _ns_gather(h, ns_p):
    mesh = plsc.VectorSubcoreMesh(core_axis_name="c", subcore_axis_name="s")
    f = pl.kernel(
        _ns_body,
        out_type=jax.ShapeDtypeStruct((_NPAD, _IN), jnp.float32),
        mesh=mesh,
        scratch_types=[
            pltpu.VMEM((_CN,), jnp.int32),
            pltpu.VMEM((_CN, _IN), jnp.float32),
            pltpu.SemaphoreType.DMA,
        ],
        compiler_params=pltpu.CompilerParams(needs_layout_passes=False),
    )
    return f(h, ns_p)


def _out_body(nsh_ref, agg_ref, w_ref, wb_ref, o_ref):
    w = w_ref[...]
    x = lax.dot_general(nsh_ref[...], w[:, :_IN], (((1,), (1,)), ((), ())),
                        preferred_element_type=jnp.float32)
    x = x + lax.dot_general(agg_ref[...], w[:, _IN:], (((1,), (1,)), ((), ())),
                            preferred_element_type=jnp.float32)
    x = jnp.maximum(x + wb_ref[...], 0.0)
    nrm = jnp.sqrt(jnp.sum(x * x, axis=1, keepdims=True))
    o_ref[...] = x / nrm


def _compute_out(nsh, agg, W_w, W_b, n):
    blk = 2000
    return pl.pallas_call(
        _out_body,
        grid=(n // blk,),
        in_specs=[pl.BlockSpec((blk, _IN), lambda i: (i, 0)),
                  pl.BlockSpec((blk, _IN), lambda i: (i, 0)),
                  pl.BlockSpec((_IN, 2 * _IN), lambda i: (0, 0)),
                  pl.BlockSpec((1, _IN), lambda i: (0, 0))],
        out_specs=pl.BlockSpec((blk, _IN), lambda i: (i, 0)),
        out_shape=jax.ShapeDtypeStruct((n, _IN), jnp.float32),
    )(nsh, agg, W_w, W_b.reshape(1, _IN))


def kernel(h, nodeset, nb_nodes, nb_weights, Q_w, Q_b, W_w, W_b):
    n_set = nodeset.shape[0]
    hq = _compute_hq(h, Q_w, Q_b)
    pad = _NPAD - n_set
    epad = pad + _NMAX - min(_N0, _N1)
    ns_p = jnp.concatenate(
        [nodeset.astype(jnp.int32), jnp.zeros((pad,), jnp.int32)])
    nb_flat = jnp.concatenate(
        [nb_nodes.astype(jnp.int32),
         jnp.zeros((epad, _T), jnp.int32)]).reshape(-1)
    w_flat = jnp.concatenate(
        [nb_weights, jnp.ones((epad, _T), jnp.float32)]).reshape(-1)
    nsh = _ns_gather(h, ns_p)
    agg = _sc_aggregate(hq, nb_flat, w_flat)
    return _compute_out(nsh, agg, W_w, W_b, n_set)


# speedup vs baseline: 1.2523x; 1.0154x over previous
import functools

import jax
import jax.numpy as jnp
import numpy as np
from jax import lax
from jax.experimental import pallas as pl
from jax.experimental.pallas import tpu as pltpu
from jax.experimental.pallas import tpu_sc as plsc

_IN = 128
_T = 32
_NPS = 640
_NPAD = 16 * _NPS
_N0 = 424
_N1 = _NPS - _N0
_CE = 64
_NPC = _CE // _T
_NBUF = 2
_CN = 80
_NMAX = max(_N0, _N1)


def _hq_body(h_ref, qw_ref, qb_ref, o_ref):
    acc = lax.dot_general(h_ref[...], qw_ref[...], (((1,), (1,)), ((), ())),
                          preferred_element_type=jnp.float32)
    o_ref[...] = jnp.maximum(acc + qb_ref[...], 0.0)


def _compute_hq(h, Q_w, Q_b):
    n = h.shape[0]
    blk = 10000
    return pl.pallas_call(
        _hq_body,
        grid=(n // blk,),
        in_specs=[pl.BlockSpec((blk, _IN), lambda i: (i, 0)),
                  pl.BlockSpec((_IN, _IN), lambda i: (0, 0)),
                  pl.BlockSpec((1, _IN), lambda i: (0, 0))],
        out_specs=pl.BlockSpec((blk, _IN), lambda i: (i, 0)),
        out_shape=jax.ShapeDtypeStruct((n, _IN), jnp.float32),
    )(h, Q_w, Q_b.reshape(1, _IN))


def _sc_body(hq_hbm, nb_hbm, w_hbm, agg_hbm,
             nb_v, ew_v, rows_v, agg_v, sem0, sem1):
    cid = lax.axis_index("c")
    sid = lax.axis_index("s")
    npw = jnp.where(cid == 0, _N0, _N1)
    nbase = sid * _NPS + cid * _N0
    ebase = nbase * _T
    nchunk = npw * _T // _CE
    sems = (sem0, sem1)

    pltpu.sync_copy(nb_hbm.at[pl.ds(ebase, _NMAX * _T)], nb_v)
    pltpu.sync_copy(w_hbm.at[pl.ds(ebase, _NMAX * _T)], ew_v)

    def start(c, b):
        pltpu.async_copy(hq_hbm.at[nb_v.at[pl.ds(c * _CE, _CE)]],
                         rows_v.at[b], sems[b], priority=b % 2)

    for b0 in range(_NBUF):
        start(b0, b0)

    def process(c, b):
        pltpu.make_async_copy(hq_hbm.at[pl.ds(0, _CE)],
                              rows_v.at[b], sems[b]).wait()
        for j in range(_NPC):
            acc = [jnp.zeros((16,), jnp.float32) for _ in range(8)]
            for t in range(_T):
                e = j * _T + t
                bw = plsc.load_gather(
                    ew_v, [jnp.full((16,), c * _CE + e, jnp.int32)])
                for f in range(8):
                    acc[f] = acc[f] + bw * rows_v[b, e, pl.ds(f * 16, 16)]
            w0 = ew_v[pl.ds(c * _CE + j * _T, 16)]
            w1 = ew_v[pl.ds(c * _CE + j * _T + 16, 16)]
            winv = 1.0 / jnp.broadcast_to(jnp.sum(w0 + w1), (16,))
            for f in range(8):
                agg_v[c * _NPC + j, pl.ds(f * 16, 16)] = acc[f] * winv

    def body(cb, carry):
        c0 = cb * _NBUF
        for b in range(_NBUF):
            c = c0 + b
            process(c, b)

            @pl.when(c + _NBUF < nchunk)
            def _():
                start(c + _NBUF, b)
        return carry

    lax.fori_loop(0, nchunk // _NBUF, body, 0)

    @pl.when(cid == 0)
    def _():
        pltpu.sync_copy(agg_v.at[pl.ds(0, _N0)], agg_hbm.at[pl.ds(nbase, _N0)])

    @pl.when(cid == 1)
    def _():
        pltpu.sync_copy(agg_v.at[pl.ds(0, _N1)], agg_hbm.at[pl.ds(nbase, _N1)])


def _sc_aggregate(hq, nb_flat, w_flat):
    mesh = plsc.VectorSubcoreMesh(core_axis_name="c", subcore_axis_name="s")
    f = pl.kernel(
        _sc_body,
        out_type=jax.ShapeDtypeStruct((_NPAD, _IN), jnp.float32),
        mesh=mesh,
        scratch_types=[
            pltpu.VMEM((_NMAX * _T,), jnp.int32),
            pltpu.VMEM((_NMAX * _T,), jnp.float32),
            pltpu.VMEM((_NBUF, _CE, _IN), jnp.float32),
            pltpu.VMEM((_NMAX, _IN), jnp.float32),
            pltpu.SemaphoreType.DMA,
            pltpu.SemaphoreType.DMA,
        ],
        compiler_params=pltpu.CompilerParams(needs_layout_passes=False),
    )
    return f(hq, nb_flat, w_flat)


def _ns_body(h_hbm, ns_hbm, nsh_hbm, nidx_v, nrows_v, semn):
    wid = lax.axis_index("s") * 2 + lax.axis_index("c")
    nsym = _NPAD // 32

    def ns_chunk(k, carry):
        noff = wid * nsym + k * _CN
        pltpu.sync_copy(ns_hbm.at[pl.ds(noff, _CN)], nidx_v)
        pltpu.async_copy(h_hbm.at[nidx_v], nrows_v, semn).wait()
        pltpu.sync_copy(nrows_v, nsh_hbm.at[pl.ds(noff, _CN)])
        return carry

    lax.fori_loop(0, nsym // _CN, ns_chunk, 0)


def _ns_gather(h, ns_p):
    mesh = plsc.VectorSubcoreMesh(core_axis_name="c", subcore_axis_name="s")
    f = pl.kernel(
        _ns_body,
        out_type=jax.ShapeDtypeStruct((_NPAD, _IN), jnp.float32),
        mesh=mesh,
        scratch_types=[
            pltpu.VMEM((_CN,), jnp.int32),
            pltpu.VMEM((_CN, _IN), jnp.float32),
            pltpu.SemaphoreType.DMA,
        ],
        compiler_params=pltpu.CompilerParams(needs_layout_passes=False),
    )
    return f(h, ns_p)


def _out_body(nsh_ref, agg_ref, w_ref, wb_ref, o_ref):
    w = w_ref[...]
    x = lax.dot_general(nsh_ref[...], w[:, :_IN], (((1,), (1,)), ((), ())),
                        preferred_element_type=jnp.float32)
    x = x + lax.dot_general(agg_ref[...], w[:, _IN:], (((1,), (1,)), ((), ())),
                            preferred_element_type=jnp.float32)
    x = jnp.maximum(x + wb_ref[...], 0.0)
    nrm = jnp.sqrt(jnp.sum(x * x, axis=1, keepdims=True))
    o_ref[...] = x / nrm


def _compute_out(nsh, agg, W_w, W_b, n):
    blk = 5000
    return pl.pallas_call(
        _out_body,
        grid=(n // blk,),
        in_specs=[pl.BlockSpec((blk, _IN), lambda i: (i, 0)),
                  pl.BlockSpec((blk, _IN), lambda i: (i, 0)),
                  pl.BlockSpec((_IN, 2 * _IN), lambda i: (0, 0)),
                  pl.BlockSpec((1, _IN), lambda i: (0, 0))],
        out_specs=pl.BlockSpec((blk, _IN), lambda i: (i, 0)),
        out_shape=jax.ShapeDtypeStruct((n, _IN), jnp.float32),
    )(nsh, agg, W_w, W_b.reshape(1, _IN))


def kernel(h, nodeset, nb_nodes, nb_weights, Q_w, Q_b, W_w, W_b):
    n_set = nodeset.shape[0]
    hq = _compute_hq(h, Q_w, Q_b)
    pad = _NPAD - n_set
    epad = pad + _NMAX - min(_N0, _N1)
    ns_p = jnp.concatenate(
        [nodeset.astype(jnp.int32), jnp.zeros((pad,), jnp.int32)])
    nb_flat = jnp.concatenate(
        [nb_nodes.astype(jnp.int32),
         jnp.zeros((epad, _T), jnp.int32)]).reshape(-1)
    w_flat = jnp.concatenate(
        [nb_weights, jnp.ones((epad, _T), jnp.float32)]).reshape(-1)
    nsh = _ns_gather(h, ns_p)
    agg = _sc_aggregate(hq, nb_flat, w_flat)
    return _compute_out(nsh, agg, W_w, W_b, n_set)


# speedup vs baseline: 1.2552x; 1.0023x over previous
import functools

import jax
import jax.numpy as jnp
import numpy as np
from jax import lax
from jax.experimental import pallas as pl
from jax.experimental.pallas import tpu as pltpu
from jax.experimental.pallas import tpu_sc as plsc

_IN = 128
_T = 32
_NPS = 640
_NPAD = 16 * _NPS
_N0 = 416
_N1 = _NPS - _N0
_CE = 64
_NPC = _CE // _T
_NBUF = 2
_CN = 80
_NMAX = max(_N0, _N1)


def _hq_body(h_ref, qw_ref, qb_ref, o_ref):
    acc = lax.dot_general(h_ref[...], qw_ref[...], (((1,), (1,)), ((), ())),
                          preferred_element_type=jnp.float32)
    o_ref[...] = jnp.maximum(acc + qb_ref[...], 0.0)


def _compute_hq(h, Q_w, Q_b):
    n = h.shape[0]
    blk = 10000
    return pl.pallas_call(
        _hq_body,
        grid=(n // blk,),
        in_specs=[pl.BlockSpec((blk, _IN), lambda i: (i, 0)),
                  pl.BlockSpec((_IN, _IN), lambda i: (0, 0)),
                  pl.BlockSpec((1, _IN), lambda i: (0, 0))],
        out_specs=pl.BlockSpec((blk, _IN), lambda i: (i, 0)),
        out_shape=jax.ShapeDtypeStruct((n, _IN), jnp.float32),
    )(h, Q_w, Q_b.reshape(1, _IN))


def _sc_body(hq_hbm, nb_hbm, w_hbm, agg_hbm,
             nb_v, ew_v, rows_v, agg_v, sem0, sem1):
    cid = lax.axis_index("c")
    sid = lax.axis_index("s")
    npw = jnp.where(cid == 0, _N0, _N1)
    nbase = sid * _NPS + cid * _N0
    ebase = nbase * _T
    nchunk = npw * _T // _CE
    sems = (sem0, sem1)

    pltpu.sync_copy(nb_hbm.at[pl.ds(ebase, _NMAX * _T)], nb_v)
    pltpu.sync_copy(w_hbm.at[pl.ds(ebase, _NMAX * _T)], ew_v)

    def start(c, b):
        pltpu.async_copy(hq_hbm.at[nb_v.at[pl.ds(c * _CE, _CE)]],
                         rows_v.at[b], sems[b], priority=b % 2)

    for b0 in range(_NBUF):
        start(b0, b0)

    def process(c, b):
        pltpu.make_async_copy(hq_hbm.at[pl.ds(0, _CE)],
                              rows_v.at[b], sems[b]).wait()
        for j in range(_NPC):
            acc = [jnp.zeros((16,), jnp.float32) for _ in range(8)]
            for t in range(_T):
                e = j * _T + t
                bw = plsc.load_gather(
                    ew_v, [jnp.full((16,), c * _CE + e, jnp.int32)])
                for f in range(8):
                    acc[f] = acc[f] + bw * rows_v[b, e, pl.ds(f * 16, 16)]
            w0 = ew_v[pl.ds(c * _CE + j * _T, 16)]
            w1 = ew_v[pl.ds(c * _CE + j * _T + 16, 16)]
            winv = 1.0 / jnp.broadcast_to(jnp.sum(w0 + w1), (16,))
            for f in range(8):
                agg_v[c * _NPC + j, pl.ds(f * 16, 16)] = acc[f] * winv

    def body(cb, carry):
        c0 = cb * _NBUF
        for b in range(_NBUF):
            c = c0 + b
            process(c, b)

            @pl.when(c + _NBUF < nchunk)
            def _():
                start(c + _NBUF, b)
        return carry

    lax.fori_loop(0, nchunk // _NBUF, body, 0)

    @pl.when(cid == 0)
    def _():
        pltpu.sync_copy(agg_v.at[pl.ds(0, _N0)], agg_hbm.at[pl.ds(nbase, _N0)])

    @pl.when(cid == 1)
    def _():
        pltpu.sync_copy(agg_v.at[pl.ds(0, _N1)], agg_hbm.at[pl.ds(nbase, _N1)])


def _sc_aggregate(hq, nb_flat, w_flat):
    mesh = plsc.VectorSubcoreMesh(core_axis_name="c", subcore_axis_name="s")
    f = pl.kernel(
        _sc_body,
        out_type=jax.ShapeDtypeStruct((_NPAD, _IN), jnp.float32),
        mesh=mesh,
        scratch_types=[
            pltpu.VMEM((_NMAX * _T,), jnp.int32),
            pltpu.VMEM((_NMAX * _T,), jnp.float32),
            pltpu.VMEM((_NBUF, _CE, _IN), jnp.float32),
            pltpu.VMEM((_NMAX, _IN), jnp.float32),
            pltpu.SemaphoreType.DMA,
            pltpu.SemaphoreType.DMA,
        ],
        compiler_params=pltpu.CompilerParams(needs_layout_passes=False),
    )
    return f(hq, nb_flat, w_flat)


def _ns_body(h_hbm, ns_hbm, nsh_hbm, nidx_v, nrows_v, semn):
    cid = lax.axis_index("c")
    sid = lax.axis_index("s")
    n0 = 400
    npw = jnp.where(cid == 0, n0, _NPS - n0)
    base = sid * _NPS + cid * n0

    def ns_chunk(k, carry):
        noff = base + k * _CN
        pltpu.sync_copy(ns_hbm.at[pl.ds(noff, _CN)], nidx_v)
        pltpu.async_copy(h_hbm.at[nidx_v], nrows_v, semn).wait()
        pltpu.sync_copy(nrows_v, nsh_hbm.at[pl.ds(noff, _CN)])
        return carry

    lax.fori_loop(0, npw // _CN, ns_chunk, 0)


def _ns_gather(h, ns_p):
    mesh = plsc.VectorSubcoreMesh(core_axis_name="c", subcore_axis_name="s")
    f = pl.kernel(
        _ns_body,
        out_type=jax.ShapeDtypeStruct((_NPAD, _IN), jnp.float32),
        mesh=mesh,
        scratch_types=[
            pltpu.VMEM((_CN,), jnp.int32),
            pltpu.VMEM((_CN, _IN), jnp.float32),
            pltpu.SemaphoreType.DMA,
        ],
        compiler_params=pltpu.CompilerParams(needs_layout_passes=False),
    )
    return f(h, ns_p)


def _out_body(nsh_ref, agg_ref, w_ref, wb_ref, o_ref):
    w = w_ref[...]
    x = lax.dot_general(nsh_ref[...], w[:, :_IN], (((1,), (1,)), ((), ())),
                        preferred_element_type=jnp.float32)
    x = x + lax.dot_general(agg_ref[...], w[:, _IN:], (((1,), (1,)), ((), ())),
                            preferred_element_type=jnp.float32)
    x = jnp.maximum(x + wb_ref[...], 0.0)
    nrm = jnp.sqrt(jnp.sum(x * x, axis=1, keepdims=True))
    o_ref[...] = x / nrm


def _compute_out(nsh, agg, W_w, W_b, n):
    blk = 5000
    return pl.pallas_call(
        _out_body,
        grid=(n // blk,),
        in_specs=[pl.BlockSpec((blk, _IN), lambda i: (i, 0)),
                  pl.BlockSpec((blk, _IN), lambda i: (i, 0)),
                  pl.BlockSpec((_IN, 2 * _IN), lambda i: (0, 0)),
                  pl.BlockSpec((1, _IN), lambda i: (0, 0))],
        out_specs=pl.BlockSpec((blk, _IN), lambda i: (i, 0)),
        out_shape=jax.ShapeDtypeStruct((n, _IN), jnp.float32),
    )(nsh, agg, W_w, W_b.reshape(1, _IN))


def kernel(h, nodeset, nb_nodes, nb_weights, Q_w, Q_b, W_w, W_b):
    n_set = nodeset.shape[0]
    hq = _compute_hq(h, Q_w, Q_b)
    pad = _NPAD - n_set
    epad = pad + _NMAX - min(_N0, _N1)
    ns_p = jnp.concatenate(
        [nodeset.astype(jnp.int32), jnp.zeros((pad,), jnp.int32)])
    nb_flat = jnp.concatenate(
        [nb_nodes.astype(jnp.int32),
         jnp.zeros((epad, _T), jnp.int32)]).reshape(-1)
    w_flat = jnp.concatenate(
        [nb_weights, jnp.ones((epad, _T), jnp.float32)]).reshape(-1)
    nsh = _ns_gather(h, ns_p)
    agg = _sc_aggregate(hq, nb_flat, w_flat)
    return _compute_out(nsh, agg, W_w, W_b, n_set)
